# Initial kernel scaffold; baseline (speedup 1.0000x reference)
import jax
import jax.numpy as jnp
from jax.experimental import pallas as pl


def kernel(x, edge_index, W, b):
    raise NotImplementedError("write your pallas kernel here")


# speedup vs baseline: 32.9745x; 32.9745x over previous
import jax
import jax.numpy as jnp
from jax.experimental import pallas as pl
from jax.experimental.pallas import tpu as pltpu

BZ = 256
C = 128
V = 256
E = 2048


def _body(e_ref, x_ref, w_ref, b_ref, o_ref):
    y = jnp.dot(w_ref[...], x_ref[0], preferred_element_type=jnp.float32,
                precision=jax.lax.Precision.HIGHEST) + b_ref[...]
    l = e_ref[0, 0, :]
    r = e_ref[0, 1, :]
    iv = jax.lax.broadcasted_iota(jnp.int32, (V, E), 0)
    loh = (iv == l[None, :]).astype(jnp.bfloat16)
    rt = (jax.lax.broadcasted_iota(jnp.int32, (E, V), 1)
          == r[:, None]).astype(jnp.bfloat16)
    a = jnp.dot(loh, rt, preferred_element_type=jnp.float32)
    o_ref[0] = jnp.dot(a, y.T, preferred_element_type=jnp.float32,
                       precision=jax.lax.Precision.HIGHEST)


def kernel(x, edge_index, W, b):
    edge_index = edge_index.astype(jnp.int32)
    return pl.pallas_call(
        _body,
        grid=(BZ,),
        in_specs=[
            pl.BlockSpec((1, 2, E), lambda i: (i, 0, 0)),
            pl.BlockSpec((1, C, V), lambda i: (i, 0, 0)),
            pl.BlockSpec((C, C), lambda i: (0, 0)),
            pl.BlockSpec((C, 1), lambda i: (0, 0)),
        ],
        out_specs=pl.BlockSpec((1, V, C), lambda i: (i, 0, 0)),
        out_shape=jax.ShapeDtypeStruct((BZ, V, C), jnp.float32),
        compiler_params=pltpu.CompilerParams(
            dimension_semantics=("parallel",)),
    )(edge_index, x, W, b.reshape(C, 1))
